# Initial kernel scaffold; baseline (speedup 1.0000x reference)
import jax
import jax.numpy as jnp
from jax.experimental import pallas as pl


def kernel(predict_seg_map, real_label):
    raise NotImplementedError("write your pallas kernel here")


# speedup vs baseline: 58.3231x; 58.3231x over previous
import functools

import numpy as np

import jax
import jax.numpy as jnp
from jax import lax
from jax.experimental import pallas as pl
from jax.experimental.pallas import tpu as pltpu
from jax.experimental.pallas import tpu_sc as plsc

_N_POS = 128
_N_NEG = 512
_NSEL = _N_POS + _N_NEG
_B, _C, _H, _W = 4, 96, 512, 512
_HW = _H * _W
_NPIX = _B * _HW
_CHW = _C * _HW
_TEMP = 0.1
_EPS = 1e-6

_NC, _NS = 2, 16
_NW = _NC * _NS
_SPW = _NSEL // _NW
_GPW = _SPW * _C
_GCH = 128
_CH = 2048
_CPAD = 128


def _threefry2x32_pair(k1, k2, x0, x1):
    x = [x0.astype(np.uint32).copy(), x1.astype(np.uint32).copy()]
    rotations = [(13, 15, 26, 6), (17, 29, 16, 24)]
    k1 = np.uint32(k1)
    k2 = np.uint32(k2)
    ks = [k1, k2, k1 ^ k2 ^ np.uint32(0x1BD11BDA)]

    def rotl(v, d):
        return (v << np.uint32(d)) | (v >> np.uint32(32 - d))

    x[0] = x[0] + ks[0]
    x[1] = x[1] + ks[1]
    kidx = [1, 2, 0, 1, 2, 0]
    for i in range(5):
        for r in rotations[i % 2]:
            x[0] = x[0] + x[1]
            x[1] = rotl(x[1], r)
            x[1] = x[0] ^ x[1]
        x[0] = x[0] + ks[kidx[i]]
        x[1] = x[1] + ks[kidx[i + 1]] + np.uint32(i + 1)
    return x[0], x[1]


def _np_uniform_bits(k1, k2, n):
    b1, b2 = _threefry2x32_pair(k1, k2, np.zeros(n, np.uint32),
                                np.arange(n, dtype=np.uint32))
    bits = b1 ^ b2
    f = ((bits >> np.uint32(9)) | np.uint32(0x3F800000)).view(np.float32) - 1.0
    return np.maximum(np.float32(0.0), f).astype(np.float32)


def _compute_orders():
    b1, b2 = _threefry2x32_pair(np.uint32(0), np.uint32(1234),
                                np.zeros(2, np.uint32),
                                np.arange(2, dtype=np.uint32))
    kp = (b1[0], b2[0])
    kn = (b1[1], b2[1])
    sp = _np_uniform_bits(kp[0], kp[1], _NPIX)
    sn = _np_uniform_bits(kn[0], kn[1], _NPIX)
    order_pos = np.argsort(-sp, kind="stable").astype(np.int64)
    order_neg = np.argsort(-sn, kind="stable").astype(np.int64)

    def tiled(p):
        b = p >> 18
        hw = p & (_HW - 1)
        t = (((hw >> 12) << 12) | (((hw >> 7) & 3) << 10)
             | (((hw >> 9) & 7) << 7) | (hw & 127))
        return ((b << 18) | t).astype(np.int32)

    return tiled(order_pos), tiled(order_neg)


_ORDER_POS, _ORDER_NEG = _compute_orders()


def _sc_select_gather_body(feat_hbm, opos_hbm, oneg_hbm, label_hbm, out_hbm,
                           ord_v, lab_v, selbase_v, idx_v, gath_v, shr_v, sem):
    cid = lax.axis_index("c")
    sid = lax.axis_index("s")
    wid = sid * _NC + cid

    def select(order_hbm, n_sel, sel_off, want_pos):
        def chunk_body(chunk, off_c):
            base_r = pl.multiple_of(chunk * _CH, _CH)
            pltpu.sync_copy(order_hbm.at[pl.ds(base_r, _CH)], ord_v)
            copies = []
            for j in range(_CH // _GCH):
                copies.append(pltpu.async_copy(
                    label_hbm.at[ord_v.at[pl.ds(j * _GCH, _GCH)]],
                    lab_v.at[pl.ds(j * _GCH, _GCH)], sem))
            for cp in copies:
                cp.wait()

            def group(g, off_i):
                lab = lab_v[pl.ds(g * 16, 16)]
                vals = ord_v[pl.ds(g * 16, 16)]
                if want_pos:
                    m = lab > 0.5
                else:
                    m = lab <= 0.5
                mi = m.astype(jnp.int32)
                pos = off_i + plsc.cumsum(mi) - 1
                wm = jnp.logical_and(m, pos < n_sel)
                base = ((vals >> 18) * _CHW) + (vals & (_HW - 1))
                plsc.store_scatter(selbase_v, [pos + sel_off], base,
                                   mask=wm)
                return off_i + jnp.sum(mi)

            def guarded_group(g, off_i):
                return lax.cond(off_i < n_sel,
                                lambda o: group(g, o), lambda o: o, off_i)

            return lax.fori_loop(0, _CH // 16, guarded_group, off_c)

        def guarded_chunk(chunk, off):
            return lax.cond(off < n_sel,
                            lambda o: chunk_body(chunk, o), lambda o: o, off)

        off0 = chunk_body(0, jnp.int32(0))
        lax.cond(off0 < n_sel,
                 lambda o: lax.fori_loop(1, _NPIX // _CH, guarded_chunk, o),
                 lambda o: o, off0)

    @pl.when(sid == 0)
    def _():
        select(opos_hbm, _N_POS, 0, True)
        pltpu.sync_copy(selbase_v.at[pl.ds(0, _N_POS)],
                        shr_v.at[pl.ds(0, _N_POS)])

    @pl.when(sid == 1)
    def _():
        select(oneg_hbm, _N_NEG, _N_POS, False)
        pltpu.sync_copy(selbase_v.at[pl.ds(_N_POS, _N_NEG)],
                        shr_v.at[pl.ds(_N_POS, _N_NEG)])

    plsc.subcore_barrier()
    pltpu.sync_copy(shr_v, selbase_v)

    base_row = wid * _SPW
    chan_off = lax.iota(jnp.int32, 16) * _HW

    def build(j, carry):
        samp = jnp.full((16,), 0, jnp.int32) + (base_row + j)
        sbase = plsc.load_gather(selbase_v, [samp])
        for g in range(_C // 16):
            idx_v[pl.ds(j * _C + g * 16, 16)] = (
                sbase + (g * 16) * _HW + chan_off)
        return carry

    lax.fori_loop(0, _SPW, build, jnp.int32(0))

    copies = []
    for j in range(_SPW):
        copies.append(pltpu.async_copy(
            feat_hbm.at[idx_v.at[pl.ds(j * _C, _C)]],
            gath_v.at[pl.ds(j * _CPAD, _C)], sem))
    for cp in copies:
        cp.wait()

    out_off = pl.multiple_of(wid * (_SPW * _CPAD), _SPW * _CPAD)
    pltpu.sync_copy(gath_v, out_hbm.at[pl.ds(out_off, _SPW * _CPAD)])


_sc_select_gather = functools.partial(
    pl.kernel,
    out_type=jax.ShapeDtypeStruct((_NSEL * _CPAD,), jnp.float32),
    mesh=plsc.VectorSubcoreMesh(core_axis_name="c", subcore_axis_name="s"),
    compiler_params=pltpu.CompilerParams(needs_layout_passes=False),
    scratch_types=[
        pltpu.VMEM((_CH,), jnp.int32),
        pltpu.VMEM((_CH,), jnp.float32),
        pltpu.VMEM((_NSEL,), jnp.int32),
        pltpu.VMEM((_GPW,), jnp.int32),
        pltpu.VMEM((_SPW * _CPAD,), jnp.float32),
        pltpu.VMEM_SHARED((_NSEL,), jnp.int32),
        pltpu.SemaphoreType.DMA,
    ],
)(_sc_select_gather_body)


def _loss_body(sel_ref, out_ref):
    pos = sel_ref[0:_N_POS, 0:_C]
    neg = sel_ref[_N_POS:_NSEL, 0:_C]
    pn = pos / jnp.maximum(
        jnp.sqrt(jnp.sum(pos * pos, axis=1, keepdims=True)), _EPS)
    nn = neg / jnp.maximum(
        jnp.sqrt(jnp.sum(neg * neg, axis=1, keepdims=True)), _EPS)
    ps = lax.dot_general(pn, pn, (((1,), (1,)), ((), ())),
                         preferred_element_type=jnp.float32)
    ns = lax.dot_general(pn, nn, (((1,), (1,)), ((), ())),
                         preferred_element_type=jnp.float32)
    prs = jnp.sum(jnp.exp(ps / _TEMP), axis=1) - np.float32(np.exp(1.0))
    nrs = jnp.sum(jnp.exp(ns / _TEMP), axis=1)
    lik = prs / (prs + nrs)
    nll = -jnp.mean(jnp.log(lik))
    out_ref[0, 0] = nll


_loss_call = pl.pallas_call(
    _loss_body,
    out_shape=jax.ShapeDtypeStruct((1, 1), jnp.float32),
    out_specs=pl.BlockSpec(memory_space=pltpu.SMEM),
)


def kernel(predict_seg_map, real_label):
    feat = (predict_seg_map
            .reshape(_B * _C * (_H // 8), 8, _W // 128, 128)
            .transpose(0, 2, 1, 3)
            .reshape(-1))
    label = (real_label
             .reshape(_B * (_H // 8), 8, _W // 128, 128)
             .transpose(0, 2, 1, 3)
             .reshape(-1))
    gathered = _sc_select_gather(
        feat, jnp.asarray(_ORDER_POS), jnp.asarray(_ORDER_NEG), label)
    sel = gathered.reshape(_NSEL, _CPAD)
    return _loss_call(sel)[0, 0]
